# baseline (speedup 1.0000x reference)
import functools

import jax
import jax.numpy as jnp
from jax import lax
from jax.experimental import pallas as pl
from jax.experimental.pallas import tpu as pltpu
from jax.experimental.pallas import tpu_sc as plsc


def kernel(tokens, table):
    B0, S = tokens.shape
    V, D = table.shape
    B = B0 * S
    info = plsc.get_sparse_core_info()
    NC, NS = info.num_cores, info.num_subcores
    NW = NC * NS
    C = 512
    NBUF = 2
    bw = B // NW
    nch = bw // C
    R = nch // NBUF

    idx = tokens.reshape(NW, nch, C).astype(jnp.int32)

    mesh = plsc.VectorSubcoreMesh(core_axis_name="c", subcore_axis_name="s")

    @functools.partial(
        pl.kernel,
        mesh=mesh,
        compiler_params=pltpu.CompilerParams(use_tc_tiling_on_sc=False),
        out_type=jax.ShapeDtypeStruct((B, D), jnp.float32),
        scratch_types=(
            [pltpu.VMEM((nch, C), jnp.int32)]
            + [pltpu.VMEM((C, D), jnp.float32) for _ in range(NBUF)]
            + [pltpu.SemaphoreType.DMA for _ in range(2 * NBUF)]
        ),
    )
    def emb(idx_hbm, table_hbm, out_hbm, idx_v, *rest):
        bufs = rest[:NBUF]
        gsem = rest[NBUF:2 * NBUF]
        wsem = rest[2 * NBUF:]
        wid = lax.axis_index("s") * NC + lax.axis_index("c")
        base = wid * bw
        pltpu.sync_copy(idx_hbm.at[wid], idx_v)

        def gather(j, b):
            return pltpu.make_async_copy(
                table_hbm.at[idx_v.at[j]], bufs[b], gsem[b])

        def write(j, b):
            return pltpu.make_async_copy(
                bufs[b], out_hbm.at[pl.ds(base + j * C, C)], wsem[b])

        for b in range(NBUF):
            gather(b, b).start()

        def body(r, carry):
            jp = (r - 1) * NBUF
            j0 = r * NBUF
            for b in range(NBUF):
                gather(jp + b, b).wait()
                write(jp + b, b).start()
            for b in range(NBUF):
                write(jp + b, b).wait()
                gather(j0 + b, b).start()
            return carry

        lax.fori_loop(1, R, body, 0)

        jl = (R - 1) * NBUF
        for b in range(NBUF):
            gather(jl + b, b).wait()
            write(jl + b, b).start()
        for b in range(NBUF):
            write(jl + b, b).wait()

    out = emb(idx, table)
    return out.reshape(B0, S, D)


# speedup vs baseline: 1.2214x; 1.2214x over previous
import functools

import jax
import jax.numpy as jnp
from jax import lax
from jax.experimental import pallas as pl
from jax.experimental.pallas import tpu as pltpu
from jax.experimental.pallas import tpu_sc as plsc


def kernel(tokens, table):
    B0, S = tokens.shape
    V, D = table.shape
    B = B0 * S
    info = plsc.get_sparse_core_info()
    NC, NS = info.num_cores, info.num_subcores
    NW = NC * NS
    C = 128
    NBUF = 4
    bw = B // NW
    nch = bw // C
    R = nch // NBUF

    tableP = jnp.pad(table, ((0, 0), (0, 128 - D)))
    idx = tokens.reshape(NW, nch, C).astype(jnp.int32)

    mesh = plsc.VectorSubcoreMesh(core_axis_name="c", subcore_axis_name="s")

    @functools.partial(
        pl.kernel,
        mesh=mesh,
        compiler_params=pltpu.CompilerParams(use_tc_tiling_on_sc=True),
        out_type=jax.ShapeDtypeStruct((B, 128), jnp.float32),
        scratch_types=(
            [pltpu.VMEM((nch, C), jnp.int32)]
            + [pltpu.VMEM((C, 128), jnp.float32) for _ in range(NBUF)]
            + [pltpu.SemaphoreType.DMA for _ in range(2 * NBUF)]
        ),
    )
    def emb(idx_hbm, table_hbm, out_hbm, idx_v, *rest):
        bufs = rest[:NBUF]
        gsem = rest[NBUF:2 * NBUF]
        wsem = rest[2 * NBUF:]
        wid = lax.axis_index("s") * NC + lax.axis_index("c")
        base = wid * bw
        pltpu.sync_copy(idx_hbm.at[wid], idx_v)

        def gather(j, b):
            return pltpu.make_async_copy(
                table_hbm.at[idx_v.at[j]], bufs[b], gsem[b])

        def write(j, b):
            return pltpu.make_async_copy(
                bufs[b], out_hbm.at[pl.ds(base + j * C, C)], wsem[b])

        for b in range(NBUF):
            gather(b, b).start()

        def body(r, carry):
            jp = (r - 1) * NBUF
            j0 = r * NBUF
            for b in range(NBUF):
                gather(jp + b, b).wait()
                write(jp + b, b).start()
            for b in range(NBUF):
                write(jp + b, b).wait()
                gather(j0 + b, b).start()
            return carry

        lax.fori_loop(1, R, body, 0)

        jl = (R - 1) * NBUF
        for b in range(NBUF):
            gather(jl + b, b).wait()
            write(jl + b, b).start()
        for b in range(NBUF):
            write(jl + b, b).wait()

    out = emb(idx, tableP)
    return out[:, :D].reshape(B0, S, D)
